# Initial kernel scaffold; baseline (speedup 1.0000x reference)
import jax
import jax.numpy as jnp
from jax.experimental import pallas as pl


def kernel(logits, top_p, top_k, min_p, soft_mask):
    raise NotImplementedError("write your pallas kernel here")


# speedup vs baseline: 5.3977x; 5.3977x over previous
import functools

import jax
import jax.numpy as jnp
from jax import lax
from jax.experimental import pallas as pl
from jax.experimental.pallas import tpu as pltpu
from jax.experimental.pallas import tpu_sc as plsc

B = 128
V = 100000
K = 2048
M = 4096
NB = 4096
CH = 10000
NCH = V // CH
GRP = CH // 16
HKEY = 0x7FFFFFFF


def _sc_select(logits):
    info = plsc.get_sparse_core_info()
    nc, ns = info.num_cores, info.num_subcores
    nw = nc * ns
    rows_per_w = B // nw
    mesh = plsc.VectorSubcoreMesh(core_axis_name="c", subcore_axis_name="s")

    @functools.partial(
        pl.kernel,
        mesh=mesh,
        compiler_params=pltpu.CompilerParams(needs_layout_passes=False),
        out_type=[
            jax.ShapeDtypeStruct((B * M,), jnp.float32),
            jax.ShapeDtypeStruct((B * M,), jnp.int32),
            jax.ShapeDtypeStruct((B * 16,), jnp.float32),
        ],
        scratch_types=[
            pltpu.VMEM((CH,), jnp.float32),
            pltpu.VMEM((16 * NB,), jnp.int32),
            pltpu.VMEM((M,), jnp.float32),
            pltpu.VMEM((M,), jnp.int32),
            pltpu.VMEM((16,), jnp.float32),
            pltpu.SemaphoreType.DMA,
        ],
    )
    def k(logits_hbm, cv_hbm, ci_hbm, st_hbm, buf, hist, cv, ci, st, sem):
        wid = lax.axis_index("s") * nc + lax.axis_index("c")
        lanes = lax.iota(jnp.int32, 16)
        lane_stride = lanes * NB
        ones = jnp.ones((16,), jnp.int32)

        def zero_hist(i, c):
            hist[pl.ds(i * 16, 16)] = jnp.zeros((16,), jnp.int32)
            return c

        lax.fori_loop(0, 16 * NB // 16, zero_hist, 0)

        def bucket_of(x):
            bits = lax.bitcast_convert_type(x, jnp.int32)
            key = bits ^ ((bits >> 31) & HKEY)
            return (key >> 20) + 2048, key

        def row_body(r, _unused):
            row = wid * rows_per_w + r
            rowbase = row * V

            def chunk1(c, vmax):
                pltpu.sync_copy(logits_hbm.at[pl.ds(rowbase + c * CH, CH)], buf)

                def body1(g, vmax):
                    x = buf[pl.ds(g * 16, 16)]
                    bkt, _ = bucket_of(x)
                    plsc.addupdate_scatter(hist, [lane_stride + bkt], ones)
                    return jnp.maximum(vmax, x)

                return lax.fori_loop(0, GRP, body1, vmax, unroll=4)

            vmax = lax.fori_loop(
                0, NCH, chunk1, jnp.full((16,), -jnp.inf, jnp.float32))
            mx = jnp.max(vmax)

            def body2(t_rev, carry):
                acc, bnd, found = carry
                t = (NB // 16 - 1) - t_rev
                s = jnp.zeros((16,), jnp.int32)
                z = jnp.zeros((16,), jnp.int32)
                for l in range(16):
                    off = l * NB + t * 16
                    s = s + hist[pl.ds(off, 16)]
                    hist[pl.ds(off, 16)] = z
                cs = plsc.cumsum(s)
                total = jnp.max(cs)
                ssum = total - cs + s
                m = (acc + ssum) >= K
                cnt = jnp.max(plsc.all_reduce_population_count(m))
                newly = jnp.logical_and(jnp.logical_not(found), cnt > 0)
                bnd = jnp.where(newly, t * 16 + cnt - 1, bnd)
                found = jnp.logical_or(found, cnt > 0)
                return acc + total, bnd, found

            _, bnd, _ = lax.fori_loop(
                0, NB // 16, body2, (jnp.int32(0), jnp.int32(0), False))

            def body3(i, c):
                cv[pl.ds(i * 16, 16)] = jnp.full((16,), -jnp.inf, jnp.float32)
                ci[pl.ds(i * 16, 16)] = jnp.zeros((16,), jnp.int32)
                return c

            lax.fori_loop(0, M // 16, body3, 0)

            def chunk2(c, carry):
                pltpu.sync_copy(logits_hbm.at[pl.ds(rowbase + c * CH, CH)], buf)

                def body4(g, carry):
                    off, sume = carry
                    x = buf[pl.ds(g * 16, 16)]
                    bkt, _ = bucket_of(x)
                    m = bkt >= bnd
                    pcs = plsc.cumsum(jnp.where(m, 1, 0))
                    pos = off + pcs - 1
                    plsc.store_scatter(cv, [pos], x, mask=m)
                    gidx = (c * CH + g * 16) + lanes
                    plsc.store_scatter(ci, [pos], gidx, mask=m)
                    cnt = jnp.max(plsc.all_reduce_population_count(m))
                    off = jnp.minimum(off + cnt, M - 16)
                    sume = sume + jnp.exp(x - mx)
                    return off, sume

                return lax.fori_loop(0, GRP, body4, carry, unroll=4)

            off, sume = lax.fori_loop(
                0, NCH, chunk2, (jnp.int32(0), jnp.zeros((16,), jnp.float32)))
            se = jnp.sum(sume)

            stv = jnp.where(lanes == 0, mx, jnp.where(lanes == 1, se, 0.0))
            st[pl.ds(0, 16)] = stv

            pltpu.sync_copy(cv, cv_hbm.at[pl.ds(row * M, M)])
            pltpu.sync_copy(ci, ci_hbm.at[pl.ds(row * M, M)])
            pltpu.sync_copy(st, st_hbm.at[pl.ds(row * 16, 16)])
            return 0

        lax.fori_loop(0, rows_per_w, row_body, 0)

    cv, ci, st = k(logits.reshape(B * V))
    return cv.reshape(B, M), ci.reshape(B, M), st.reshape(B, 16)


BR = 8


def _tc_body(cv_ref, ci_ref, st_ref, tp_ref, tk_ref, mp_ref, sm_ref, g_ref,
             fp_ref, fi_ref, tok_ref):
    p0 = jnp.exp(cv_ref[...] - st_ref[:, 0:1]) / st_ref[:, 1:2]
    idx0 = ci_ref[...]

    lane_m = lax.broadcasted_iota(jnp.int32, (BR, M), 1)

    def stage(p, idx, k, j):
        is_lower = (lane_m & j) == 0
        desc = (lane_m & k) == 0
        pa = jnp.where(is_lower, pltpu.roll(p, M - j, 1), pltpu.roll(p, j, 1))
        ia = jnp.where(is_lower, pltpu.roll(idx, M - j, 1), pltpu.roll(idx, j, 1))
        gt = (p > pa) | ((p == pa) & (idx < ia))
        keep_self = (is_lower == desc) == gt
        return jnp.where(keep_self, p, pa), jnp.where(keep_self, idx, ia)

    def outer(kk, carry):
        k = jnp.int32(1) << kk

        def inner(t, carry):
            p, idx = carry
            j = jnp.int32(1) << (kk - 1 - t)
            return stage(p, idx, k, j)

        return lax.fori_loop(0, kk, inner, carry)

    p, idx = lax.fori_loop(1, 13, outer, (p0, idx0))

    ps = p[:, :K]
    pi = idx[:, :K]

    cs = ps
    s = 1
    while s < K:
        shifted = jnp.concatenate(
            [jnp.zeros((BR, s), jnp.float32), cs[:, :K - s]], axis=1)
        cs = cs + shifted
        s *= 2

    lane = lax.broadcasted_iota(jnp.int32, (BR, K), 1)
    tk = jnp.maximum(tk_ref[...], 1)
    tp = tp_ref[...]
    mp = mp_ref[...]
    apply_min_p = mp > 0.0
    mask_k = lane < tk
    mask_p = jnp.logical_not(cs - ps > tp)
    thr = jnp.where(apply_min_p, ps[:, 0:1] * mp, 0.0)
    fm = mask_k & mask_p
    fm = fm & jnp.logical_not(apply_min_p & (ps < thr))
    filtered = jnp.where(fm, ps, 0.0)
    denom = jnp.sum(filtered, axis=1, keepdims=True)
    denom_safe = jnp.where(denom == 0.0, 1.0, denom)
    normed = filtered / denom_safe
    normed = jnp.where((denom == 0.0) & (lane == 0), 1.0, normed)

    logp = jnp.where(normed > 0.0, jnp.log(jnp.maximum(normed, 1e-38)), -jnp.inf)
    scores = logp + g_ref[...]
    mval = jnp.max(scores, axis=1, keepdims=True)
    hit = jnp.where(scores == mval, lane, K)
    sampled = jnp.min(hit, axis=1, keepdims=True)
    onehot = lane == sampled
    tok = jnp.sum(jnp.where(onehot, pi, 0), axis=1, keepdims=True)

    std_probs = jnp.where(onehot, 1.0, 0.0)
    std_idx = jnp.where(onehot, tok, 0)
    sm = sm_ref[...] != 0
    fp_ref[...] = jnp.where(sm, normed, std_probs)
    fi_ref[...] = jnp.where(sm, pi, std_idx)
    tok_ref[...] = tok


def _tc_finish(cv, ci, st, tp, tk, mp, sm, g):
    def rows(cols):
        return pl.BlockSpec((BR, cols), lambda i: (i, 0))

    return pl.pallas_call(
        _tc_body,
        grid=(B // BR,),
        in_specs=[rows(M), rows(M), rows(16), rows(1), rows(1), rows(1),
                  rows(1), rows(K)],
        out_specs=[rows(K), rows(K), rows(1)],
        out_shape=[
            jax.ShapeDtypeStruct((B, K), jnp.float32),
            jax.ShapeDtypeStruct((B, K), jnp.int32),
            jax.ShapeDtypeStruct((B, 1), jnp.int32),
        ],
    )(cv, ci, st, tp, tk, mp, sm, g)


def kernel(logits, top_p, top_k, min_p, soft_mask):
    cv, ci, st = _sc_select(logits)
    g = jax.random.gumbel(jax.random.key(1234), (B, K), jnp.float32)
    fp, fi, tok = _tc_finish(
        cv, ci, st,
        top_p.reshape(B, 1), top_k.reshape(B, 1).astype(jnp.int32),
        min_p.reshape(B, 1), soft_mask.reshape(B, 1).astype(jnp.int32), g)
    return fp, fi, tok.reshape(B)


# speedup vs baseline: 5.7903x; 1.0727x over previous
import functools

import jax
import jax.numpy as jnp
from jax import lax
from jax.experimental import pallas as pl
from jax.experimental.pallas import tpu as pltpu
from jax.experimental.pallas import tpu_sc as plsc

B = 128
V = 100000
K = 2048
M = 4096
NB = 4096
CH = 10000
NCH = V // CH
GRP = CH // 16
HKEY = 0x7FFFFFFF


def _sc_select(logits):
    info = plsc.get_sparse_core_info()
    nc, ns = info.num_cores, info.num_subcores
    nw = nc * ns
    rows_per_w = B // nw
    mesh = plsc.VectorSubcoreMesh(core_axis_name="c", subcore_axis_name="s")

    @functools.partial(
        pl.kernel,
        mesh=mesh,
        compiler_params=pltpu.CompilerParams(needs_layout_passes=False),
        out_type=[
            jax.ShapeDtypeStruct((B * M,), jnp.float32),
            jax.ShapeDtypeStruct((B * M,), jnp.int32),
            jax.ShapeDtypeStruct((B * 16,), jnp.float32),
        ],
        scratch_types=[
            pltpu.VMEM((2 * CH,), jnp.float32),
            pltpu.VMEM((16 * NB,), jnp.int32),
            pltpu.VMEM((M,), jnp.float32),
            pltpu.VMEM((M,), jnp.int32),
            pltpu.VMEM((16,), jnp.float32),
            pltpu.SemaphoreType.DMA((2,)),
        ],
    )
    def k(logits_hbm, cv_hbm, ci_hbm, st_hbm, buf, hist, cv, ci, st, sem):
        wid = lax.axis_index("s") * nc + lax.axis_index("c")
        lanes = lax.iota(jnp.int32, 16)
        lane_stride = lanes * NB
        ones = jnp.ones((16,), jnp.int32)

        def zero_hist(i, c):
            hist[pl.ds(i * 16, 16)] = jnp.zeros((16,), jnp.int32)
            return c

        lax.fori_loop(0, 16 * NB // 16, zero_hist, 0)

        def bucket_of(x):
            bits = lax.bitcast_convert_type(x, jnp.int32)
            key = bits ^ ((bits >> 31) & HKEY)
            return (key >> 20) + 2048, key

        def start_copy(base, b):
            pltpu.make_async_copy(
                logits_hbm.at[pl.ds(base, CH)],
                buf.at[pl.ds(b * CH, CH)], sem.at[b]).start()

        def wait_copy(b):
            pltpu.make_async_copy(
                logits_hbm.at[pl.ds(0, CH)],
                buf.at[pl.ds(b * CH, CH)], sem.at[b]).wait()

        def run_pass(rowbase, group_body, init):
            start_copy(rowbase, 0)

            def pair(c2, carry):
                for b in range(2):
                    c = c2 * 2 + b

                    @pl.when(c + 1 < NCH)
                    def _():
                        start_copy(rowbase + (c + 1) * CH, 1 - b)

                    wait_copy(b)
                    carry = lax.fori_loop(
                        0, GRP, lambda g, cy: group_body(c, b, g, cy),
                        carry, unroll=4)
                return carry

            return lax.fori_loop(0, NCH // 2, pair, init)

        def row_body(r, _unused):
            row = wid * rows_per_w + r
            rowbase = row * V

            def body1(c, b, g, vmax):
                x = buf[pl.ds(b * CH + g * 16, 16)]
                bkt, _ = bucket_of(x)
                plsc.addupdate_scatter(hist, [lane_stride + bkt], ones)
                return jnp.maximum(vmax, x)

            vmax = run_pass(rowbase, body1,
                            jnp.full((16,), -jnp.inf, jnp.float32))
            mx = jnp.max(vmax)

            def body2(t_rev, carry):
                acc, bnd, found = carry
                t = (NB // 16 - 1) - t_rev
                s = jnp.zeros((16,), jnp.int32)
                z = jnp.zeros((16,), jnp.int32)
                for l in range(16):
                    off = l * NB + t * 16
                    s = s + hist[pl.ds(off, 16)]
                    hist[pl.ds(off, 16)] = z
                cs = plsc.cumsum(s)
                total = jnp.max(cs)
                ssum = total - cs + s
                m = (acc + ssum) >= K
                cnt = jnp.max(plsc.all_reduce_population_count(m))
                newly = jnp.logical_and(jnp.logical_not(found), cnt > 0)
                bnd = jnp.where(newly, t * 16 + cnt - 1, bnd)
                found = jnp.logical_or(found, cnt > 0)
                return acc + total, bnd, found

            _, bnd, _ = lax.fori_loop(
                0, NB // 16, body2, (jnp.int32(0), jnp.int32(0), False))

            def body3(i, c):
                cv[pl.ds(i * 16, 16)] = jnp.full((16,), -jnp.inf, jnp.float32)
                ci[pl.ds(i * 16, 16)] = jnp.zeros((16,), jnp.int32)
                return c

            lax.fori_loop(0, M // 16, body3, 0)

            def body4(c, b, g, carry):
                off, sume = carry
                x = buf[pl.ds(b * CH + g * 16, 16)]
                bkt, _ = bucket_of(x)
                m = bkt >= bnd
                plsc.store_compressed(cv.at[pl.ds(off, 16)], x, mask=m)
                gidx = (c * CH + g * 16) + lanes
                plsc.store_compressed(ci.at[pl.ds(off, 16)], gidx, mask=m)
                cnt = jnp.max(plsc.all_reduce_population_count(m))
                off = jnp.minimum(off + cnt, M - 16)
                sume = sume + jnp.exp(x - mx)
                return off, sume

            off, sume = run_pass(
                rowbase, body4, (jnp.int32(0), jnp.zeros((16,), jnp.float32)))
            se = jnp.sum(sume)

            stv = jnp.where(lanes == 0, mx, jnp.where(lanes == 1, se, 0.0))
            st[pl.ds(0, 16)] = stv

            pltpu.sync_copy(cv, cv_hbm.at[pl.ds(row * M, M)])
            pltpu.sync_copy(ci, ci_hbm.at[pl.ds(row * M, M)])
            pltpu.sync_copy(st, st_hbm.at[pl.ds(row * 16, 16)])
            return 0

        lax.fori_loop(0, rows_per_w, row_body, 0)

    cv, ci, st = k(logits.reshape(B * V))
    return cv.reshape(B, M), ci.reshape(B, M), st.reshape(B, 16)


BR = 8


def _tc_body(cv_ref, ci_ref, st_ref, tp_ref, tk_ref, mp_ref, sm_ref, g_ref,
             fp_ref, fi_ref, tok_ref):
    p0 = jnp.exp(cv_ref[...] - st_ref[:, 0:1]) / st_ref[:, 1:2]
    idx0 = ci_ref[...]

    lane_m = lax.broadcasted_iota(jnp.int32, (BR, M), 1)

    def stage(p, idx, k, j):
        is_lower = (lane_m & j) == 0
        desc = (lane_m & k) == 0
        pa = jnp.where(is_lower, pltpu.roll(p, M - j, 1), pltpu.roll(p, j, 1))
        ia = jnp.where(is_lower, pltpu.roll(idx, M - j, 1), pltpu.roll(idx, j, 1))
        gt = (p > pa) | ((p == pa) & (idx < ia))
        keep_self = (is_lower == desc) == gt
        return jnp.where(keep_self, p, pa), jnp.where(keep_self, idx, ia)

    def outer(kk, carry):
        k = jnp.int32(1) << kk

        def inner(t, carry):
            p, idx = carry
            j = jnp.int32(1) << (kk - 1 - t)
            return stage(p, idx, k, j)

        return lax.fori_loop(0, kk, inner, carry)

    p, idx = lax.fori_loop(1, 13, outer, (p0, idx0))

    ps = p[:, :K]
    pi = idx[:, :K]

    cs = ps
    s = 1
    while s < K:
        shifted = jnp.concatenate(
            [jnp.zeros((BR, s), jnp.float32), cs[:, :K - s]], axis=1)
        cs = cs + shifted
        s *= 2

    lane = lax.broadcasted_iota(jnp.int32, (BR, K), 1)
    tk = jnp.maximum(tk_ref[...], 1)
    tp = tp_ref[...]
    mp = mp_ref[...]
    apply_min_p = mp > 0.0
    mask_k = lane < tk
    mask_p = jnp.logical_not(cs - ps > tp)
    thr = jnp.where(apply_min_p, ps[:, 0:1] * mp, 0.0)
    fm = mask_k & mask_p
    fm = fm & jnp.logical_not(apply_min_p & (ps < thr))
    filtered = jnp.where(fm, ps, 0.0)
    denom = jnp.sum(filtered, axis=1, keepdims=True)
    denom_safe = jnp.where(denom == 0.0, 1.0, denom)
    normed = filtered / denom_safe
    normed = jnp.where((denom == 0.0) & (lane == 0), 1.0, normed)

    logp = jnp.where(normed > 0.0, jnp.log(jnp.maximum(normed, 1e-38)), -jnp.inf)
    scores = logp + g_ref[...]
    mval = jnp.max(scores, axis=1, keepdims=True)
    hit = jnp.where(scores == mval, lane, K)
    sampled = jnp.min(hit, axis=1, keepdims=True)
    onehot = lane == sampled
    tok = jnp.sum(jnp.where(onehot, pi, 0), axis=1, keepdims=True)

    std_probs = jnp.where(onehot, 1.0, 0.0)
    std_idx = jnp.where(onehot, tok, 0)
    sm = sm_ref[...] != 0
    fp_ref[...] = jnp.where(sm, normed, std_probs)
    fi_ref[...] = jnp.where(sm, pi, std_idx)
    tok_ref[...] = tok


def _tc_finish(cv, ci, st, tp, tk, mp, sm, g):
    def rows(cols):
        return pl.BlockSpec((BR, cols), lambda i: (i, 0))

    return pl.pallas_call(
        _tc_body,
        grid=(B // BR,),
        in_specs=[rows(M), rows(M), rows(16), rows(1), rows(1), rows(1),
                  rows(1), rows(K)],
        out_specs=[rows(K), rows(K), rows(1)],
        out_shape=[
            jax.ShapeDtypeStruct((B, K), jnp.float32),
            jax.ShapeDtypeStruct((B, K), jnp.int32),
            jax.ShapeDtypeStruct((B, 1), jnp.int32),
        ],
    )(cv, ci, st, tp, tk, mp, sm, g)


def kernel(logits, top_p, top_k, min_p, soft_mask):
    cv, ci, st = _sc_select(logits)
    g = jax.random.gumbel(jax.random.key(1234), (B, K), jnp.float32)
    fp, fi, tok = _tc_finish(
        cv, ci, st,
        top_p.reshape(B, 1), top_k.reshape(B, 1).astype(jnp.int32),
        min_p.reshape(B, 1), soft_mask.reshape(B, 1).astype(jnp.int32), g)
    return fp, fi, tok.reshape(B)
